# Initial kernel scaffold; baseline (speedup 1.0000x reference)
import functools

import jax
import jax.numpy as jnp
from jax import lax
from jax.experimental import pallas as pl
from jax.experimental.pallas import tpu as pltpu

EPS_ = 1e-10


def _loss_kernel(loc_ref, rm_ref, b_ref, out_ref, num_acc, den_acc, *, tile_r, hw, nb, nt):
    b = pl.program_id(0)
    t = pl.program_id(1)

    @pl.when(jnp.logical_and(b == 0, t == 0))
    def _init():
        num_acc[0] = 0.0
        den_acc[0] = 0.0

    x = rm_ref[0]
    loc = loc_ref[0, 0]
    bmap = b_ref[0]

    col = lax.broadcasted_iota(jnp.int32, (tile_r, hw), 1)
    mask = col == loc[:, None]

    denom = jnp.sum(x, axis=1)
    srm = jnp.sum(jnp.where(mask, x, 0.0), axis=1)
    sb = jnp.sum(jnp.where(mask, bmap, 0.0), axis=1)

    contrib = jnp.sum(sb * -jnp.log(EPS_ + srm / denom))
    num_acc[0] += contrib
    den_acc[0] += jnp.sum(sb)

    @pl.when(jnp.logical_and(b == nb - 1, t == nt - 1))
    def _fin():
        out_ref[0, 0] = num_acc[0] / (1.0 + den_acc[0])


def kernel(response_map, source_feature_1d_locations, boundaries):
    B, S, H, W = response_map.shape
    HW = H * W
    TILE_R = 32
    T = S // TILE_R

    rm = response_map.reshape(B, S, HW)
    bnd = boundaries.reshape(B, 1, HW)
    loc = source_feature_1d_locations.astype(jnp.int32).reshape(B * T, 1, TILE_R)

    out = pl.pallas_call(
        functools.partial(_loss_kernel, tile_r=TILE_R, hw=HW, nb=B, nt=T),
        grid=(B, T),
        in_specs=[
            pl.BlockSpec((1, 1, TILE_R), lambda b, t: (b * T + t, 0, 0)),
            pl.BlockSpec((1, TILE_R, HW), lambda b, t: (b, t, 0)),
            pl.BlockSpec((1, 1, HW), lambda b, t: (b, 0, 0)),
        ],
        out_specs=pl.BlockSpec((1, 1), lambda b, t: (0, 0)),
        out_shape=jax.ShapeDtypeStruct((1, 1), jnp.float32),
        scratch_shapes=[
            pltpu.SMEM((1,), jnp.float32),
            pltpu.SMEM((1,), jnp.float32),
        ],
    )(loc, rm, bnd)
    return out[0, 0]


# speedup vs baseline: 1.0747x; 1.0747x over previous
import functools

import jax
import jax.numpy as jnp
from jax import lax
from jax.experimental import pallas as pl
from jax.experimental.pallas import tpu as pltpu

EPS_ = 1e-10


def _loss_kernel(loc_ref, rm_ref, b_ref, out_ref, num_acc, den_acc, *, tile_r, hw, nb, nt):
    b = pl.program_id(0)
    t = pl.program_id(1)

    @pl.when(jnp.logical_and(b == 0, t == 0))
    def _init():
        num_acc[0] = 0.0
        den_acc[0] = 0.0

    x = rm_ref[0]
    loc = loc_ref[0, 0]
    bmap = b_ref[0]

    col = lax.broadcasted_iota(jnp.int32, (tile_r, hw), 1)
    mask = col == loc[:, None]

    denom = jnp.sum(x, axis=1)
    srm = jnp.sum(jnp.where(mask, x, 0.0), axis=1)
    sb = jnp.sum(jnp.where(mask, bmap, 0.0), axis=1)

    contrib = jnp.sum(sb * -jnp.log(EPS_ + srm / denom))
    num_acc[0] += contrib
    den_acc[0] += jnp.sum(sb)

    @pl.when(jnp.logical_and(b == nb - 1, t == nt - 1))
    def _fin():
        out_ref[...] = jnp.full((1, 1), num_acc[0] / (1.0 + den_acc[0]), jnp.float32)


def kernel(response_map, source_feature_1d_locations, boundaries):
    B, S, H, W = response_map.shape
    HW = H * W
    TILE_R = 32
    T = S // TILE_R

    rm = response_map.reshape(B, S, HW)
    bnd = boundaries.reshape(B, 1, HW)
    loc = source_feature_1d_locations.astype(jnp.int32).reshape(B * T, 1, TILE_R)

    out = pl.pallas_call(
        functools.partial(_loss_kernel, tile_r=TILE_R, hw=HW, nb=B, nt=T),
        grid=(B, T),
        in_specs=[
            pl.BlockSpec((1, 1, TILE_R), lambda b, t: (b * T + t, 0, 0)),
            pl.BlockSpec((1, TILE_R, HW), lambda b, t: (b, t, 0)),
            pl.BlockSpec((1, 1, HW), lambda b, t: (b, 0, 0)),
        ],
        out_specs=pl.BlockSpec((1, 1), lambda b, t: (0, 0)),
        out_shape=jax.ShapeDtypeStruct((1, 1), jnp.float32),
        scratch_shapes=[
            pltpu.SMEM((1,), jnp.float32),
            pltpu.SMEM((1,), jnp.float32),
        ],
    )(loc, rm, bnd)
    return out[0, 0]
